# baseline (speedup 1.0000x reference)
import functools

import jax
import jax.numpy as jnp
from jax import lax
from jax.experimental import pallas as pl
from jax.experimental.pallas import tpu as pltpu
from jax.experimental.pallas import tpu_sc as plsc

_MAGIC = 12582912.0


def _rne(v):
    return (v + _MAGIC) - _MAGIC


def _sc_body(x_hbm, o_hbm, buf):
    info = plsc.get_sparse_core_info()
    nc = info.num_cores
    wid = lax.axis_index("s") * nc + lax.axis_index("c")
    b = wid // 16
    t_base = (wid % 16) * 128

    pltpu.sync_copy(x_hbm.at[b, :, :, pl.ds(t_base, 128)], buf)

    def group_body(g, _):
        t0 = g * 16

        def red_body(k, carry):
            lo, hi = carry
            h, c = k // 64, k % 64
            v = buf[h, c, pl.ds(t0, 16)]
            return jnp.minimum(lo, v), jnp.maximum(hi, v)

        lo, hi = lax.fori_loop(
            0, 768, red_body,
            (jnp.full((16,), 3.0e38, jnp.float32),
             jnp.full((16,), -3.0e38, jnp.float32)),
            unroll=8,
        )
        delta = (hi - lo) / 255.0
        rinv = 1.0 / delta
        zp = _rne(-lo * rinv)

        def q_body(k, _):
            h, c = k // 64, k % 64
            v = buf[h, c, pl.ds(t0, 16)]
            xi = _rne(v * rinv) + zp
            q = jnp.clip(xi, 0.0, 255.0)
            buf[h, c, pl.ds(t0, 16)] = (q - zp) * delta
            return 0

        lax.fori_loop(0, 768, q_body, 0, unroll=8)
        return 0

    lax.fori_loop(0, 8, group_body, 0)
    pltpu.sync_copy(buf, o_hbm.at[b, :, :, pl.ds(t_base, 128)])


def _sc_call(xt):
    mesh = plsc.VectorSubcoreMesh(core_axis_name="c", subcore_axis_name="s")
    kfn = functools.partial(
        pl.kernel,
        mesh=mesh,
        out_type=jax.ShapeDtypeStruct(xt.shape, xt.dtype),
        scratch_types=[pltpu.VMEM((12, 64, 128), jnp.float32)],
        compiler_params=pltpu.CompilerParams(
            needs_layout_passes=False, use_tc_tiling_on_sc=True
        ),
    )(_sc_body)
    return kfn(xt)


def kernel(x):
    xt = jnp.transpose(x, (0, 1, 3, 2))
    return jnp.transpose(_sc_call(xt), (0, 1, 3, 2))


# speedup vs baseline: 2.4341x; 2.4341x over previous
import jax
import jax.numpy as jnp
from jax.experimental import pallas as pl

_TB = 256


def _quant_block(x_ref, o_ref):
    xb = x_ref[0]
    x_min = jnp.min(jnp.min(xb, axis=0), axis=0)
    x_max = jnp.max(jnp.max(xb, axis=0), axis=0)
    delta = (x_max - x_min) / 255.0
    rinv = 1.0 / delta
    zp = jnp.round(-x_min * rinv)
    r3 = rinv[None, None, :]
    z3 = zp[None, None, :]
    xi = jnp.round(xb * r3) + z3
    q = jnp.clip(xi, 0.0, 255.0)
    o_ref[0] = (q - z3) * delta[None, None, :]


def kernel(x):
    b, h, t, c = x.shape
    xt = jnp.transpose(x, (0, 1, 3, 2))
    out = pl.pallas_call(
        _quant_block,
        grid=(b, t // _TB),
        in_specs=[pl.BlockSpec((1, h, c, _TB), lambda i, j: (i, 0, 0, j))],
        out_specs=pl.BlockSpec((1, h, c, _TB), lambda i, j: (i, 0, 0, j)),
        out_shape=jax.ShapeDtypeStruct((b, h, c, t), x.dtype),
    )(xt)
    return jnp.transpose(out, (0, 1, 3, 2))


# speedup vs baseline: 3.8493x; 1.5814x over previous
import jax
import jax.numpy as jnp
from jax.experimental import pallas as pl

_TB = 1024


def _quant_block(x_ref, o_ref):
    xb = x_ref[0]
    x_min = jnp.min(jnp.min(xb, axis=0), axis=0)
    x_max = jnp.max(jnp.max(xb, axis=0), axis=0)
    delta = (x_max - x_min) / 255.0
    rinv = 1.0 / delta
    zp = jnp.round(-x_min * rinv)
    r3 = rinv[None, None, :]
    z3 = zp[None, None, :]
    xi = jnp.round(xb * r3) + z3
    q = jnp.clip(xi, 0.0, 255.0)
    o_ref[0] = (q - z3) * delta[None, None, :]


def kernel(x):
    b, h, t, c = x.shape
    xt = jnp.transpose(x, (0, 1, 3, 2))
    out = pl.pallas_call(
        _quant_block,
        grid=(b, t // _TB),
        in_specs=[pl.BlockSpec((1, h, c, _TB), lambda i, j: (i, 0, 0, j))],
        out_specs=pl.BlockSpec((1, h, c, _TB), lambda i, j: (i, 0, 0, j)),
        out_shape=jax.ShapeDtypeStruct((b, h, c, t), x.dtype),
    )(xt)
    return jnp.transpose(out, (0, 1, 3, 2))
